# Initial kernel scaffold; baseline (speedup 1.0000x reference)
import jax
import jax.numpy as jnp
from jax.experimental import pallas as pl


def kernel(x, Wq, c_keys, c_prime_keys, ln_g, ln_b, w_down, w_up, Wo):
    raise NotImplementedError("write your pallas kernel here")


# speedup vs baseline: 5.2415x; 5.2415x over previous
import jax
import jax.numpy as jnp
from jax.experimental import pallas as pl

D_MODEL = 384
N_HEAD = 6
HEAD_DIM = 64
SUB = 32
NUM_EXPERTS = 65536
SQRT_N = 256
K = 16
B, S = 4, 2048


def _matmul_kernel(x_ref, w_ref, o_ref):
    o_ref[:, :] = jnp.dot(x_ref[:, :], w_ref[:, :],
                          preferred_element_type=jnp.float32,
                          precision=jax.lax.Precision.HIGHEST)


def _matmul(x, w_t, block=512):
    n = x.shape[0]
    return pl.pallas_call(
        _matmul_kernel,
        grid=(n // block,),
        in_specs=[
            pl.BlockSpec((block, x.shape[1]), lambda i: (i, 0)),
            pl.BlockSpec(w_t.shape, lambda i: (0, 0)),
        ],
        out_specs=pl.BlockSpec((block, w_t.shape[1]), lambda i: (i, 0)),
        out_shape=jax.ShapeDtypeStruct((n, w_t.shape[1]), jnp.float32),
    )(x, w_t)


def kernel(x, Wq, c_keys, c_prime_keys, ln_g, ln_b, w_down, w_up, Wo):
    b, s, d = x.shape
    q = jnp.einsum('bsd,ed->bse', x, Wq).reshape(b, s, N_HEAD, HEAD_DIM)
    mu = q.mean(axis=-1, keepdims=True)
    var = q.var(axis=-1, keepdims=True)
    qn = (q - mu) / jnp.sqrt(var + 1e-5) * ln_g + ln_b
    q1 = qn[..., :SUB]
    q2 = qn[..., SUB:]
    scores1 = jnp.einsum('bshc,kc->bshk', q1, c_keys)
    scores2 = jnp.einsum('bshc,kc->bshk', q2, c_prime_keys)
    t1, i1 = jax.lax.top_k(scores1, K)
    t2, i2 = jax.lax.top_k(scores2, K)
    joint = t1[..., :, None] + t2[..., None, :]
    flat = joint.reshape(b, s, N_HEAD, K * K)
    final_scores, best_flat_idx = jax.lax.top_k(flat, K)
    row_idx = best_flat_idx // K
    col_idx = best_flat_idx % K
    real_row = jnp.take_along_axis(i1, row_idx, axis=-1)
    real_col = jnp.take_along_axis(i2, col_idx, axis=-1)
    global_indices = real_row * SQRT_N + real_col
    w_down_vals = jnp.take(w_down, global_indices, axis=0)
    w_up_vals = jnp.take(w_up, global_indices, axis=0)
    x_heads = x.reshape(b, s, N_HEAD, 1, HEAD_DIM)
    hidden = (x_heads * w_down_vals).sum(axis=-1)
    hidden = jax.nn.gelu(hidden, approximate=False)
    routing_weights = jax.nn.softmax(final_scores, axis=-1)
    hidden = hidden * routing_weights
    out_heads = (hidden[..., None] * w_up_vals).sum(axis=3)
    out = _matmul(out_heads.reshape(b * s, d), Wo.T).reshape(b, s, d)
    return out


# speedup vs baseline: 6.5561x; 1.2508x over previous
import numpy as np
import jax
import jax.numpy as jnp
from jax.experimental import pallas as pl

D_MODEL = 384
N_HEAD = 6
HEAD_DIM = 64
SUB = 32
NUM_EXPERTS = 65536
SQRT_N = 256
K = 16
B, S = 4, 2048

_HIGHEST = jax.lax.Precision.HIGHEST

_R_NP = np.zeros((16, 256), np.float32)
_T_NP = np.zeros((16, 256), np.float32)
for _a in range(16):
    for _b in range(16):
        _R_NP[_a, _a * 16 + _b] = 1.0
        _T_NP[_b, _a * 16 + _b] = 1.0

_NEG = np.float32(-3.0e38)
_BIG_I = np.int32(2 ** 30)


def _top16(s, payload=None):
    iota = jax.lax.broadcasted_iota(jnp.int32, s.shape, 1)
    work = s
    vals, idxs, pays = [], [], []
    for _ in range(16):
        m = jnp.max(work, axis=-1, keepdims=True)
        is_m = work == m
        pos = jnp.min(jnp.where(is_m, iota, _BIG_I), axis=-1, keepdims=True)
        sel = iota == pos
        vals.append(m)
        idxs.append(pos)
        if payload is not None:
            pays.append(jnp.sum(jnp.where(sel, payload, 0.0), axis=-1,
                                keepdims=True))
        work = jnp.where(sel, _NEG, work)
    vals = jnp.concatenate(vals, axis=1)
    idxs = jnp.concatenate(idxs, axis=1)
    pays = jnp.concatenate(pays, axis=1) if payload is not None else None
    return vals, idxs, pays


def _routing_kernel(x_ref, wq_ref, ck_ref, cpk_ref, g_ref, b_ref, r_ref,
                    t_ref, idx_ref, w_ref):
    xb = x_ref[:, :]
    q = jnp.dot(xb, wq_ref[:, :], preferred_element_type=jnp.float32)
    rm = r_ref[:, :]
    tm = t_ref[:, :]
    for h in range(N_HEAD):
        qh = q[:, h * HEAD_DIM:(h + 1) * HEAD_DIM]
        mu = jnp.mean(qh, axis=-1, keepdims=True)
        dq = qh - mu
        var = jnp.mean(dq * dq, axis=-1, keepdims=True)
        qn = dq / jnp.sqrt(var + 1e-5) * g_ref[0, :] + b_ref[0, :]
        s1 = jnp.dot(qn[:, :SUB], ck_ref[:, :],
                     preferred_element_type=jnp.float32)
        s2 = jnp.dot(qn[:, SUB:], cpk_ref[:, :],
                     preferred_element_type=jnp.float32)
        t1, i1, _ = _top16(s1)
        t2, i2, _ = _top16(s2)
        jv = (jnp.dot(t1, rm, preferred_element_type=jnp.float32,
                      precision=_HIGHEST)
              + jnp.dot(t2, tm, preferred_element_type=jnp.float32,
                        precision=_HIGHEST))
        jc = (jnp.dot(i1.astype(jnp.float32) * 256.0, rm,
                      preferred_element_type=jnp.float32, precision=_HIGHEST)
              + jnp.dot(i2.astype(jnp.float32), tm,
                        preferred_element_type=jnp.float32,
                        precision=_HIGHEST))
        fs, _, code = _top16(jv, payload=jc)
        m = jnp.max(fs, axis=-1, keepdims=True)
        e = jnp.exp(fs - m)
        w = e / jnp.sum(e, axis=-1, keepdims=True)
        idx_ref[:, h * K:(h + 1) * K] = code.astype(jnp.int32)
        w_ref[:, h * K:(h + 1) * K] = w


def _routing(x2d, Wq, c_keys, c_prime_keys, ln_g, ln_b, block=512):
    n = x2d.shape[0]
    grid = (n // block,)
    return pl.pallas_call(
        _routing_kernel,
        grid=grid,
        in_specs=[
            pl.BlockSpec((block, D_MODEL), lambda i: (i, 0)),
            pl.BlockSpec((D_MODEL, D_MODEL), lambda i: (0, 0)),
            pl.BlockSpec((SUB, SQRT_N), lambda i: (0, 0)),
            pl.BlockSpec((SUB, SQRT_N), lambda i: (0, 0)),
            pl.BlockSpec((1, HEAD_DIM), lambda i: (0, 0)),
            pl.BlockSpec((1, HEAD_DIM), lambda i: (0, 0)),
            pl.BlockSpec((16, 256), lambda i: (0, 0)),
            pl.BlockSpec((16, 256), lambda i: (0, 0)),
        ],
        out_specs=[
            pl.BlockSpec((block, N_HEAD * K), lambda i: (i, 0)),
            pl.BlockSpec((block, N_HEAD * K), lambda i: (i, 0)),
        ],
        out_shape=[
            jax.ShapeDtypeStruct((n, N_HEAD * K), jnp.int32),
            jax.ShapeDtypeStruct((n, N_HEAD * K), jnp.float32),
        ],
    )(x2d, Wq.T, c_keys.T, c_prime_keys.T, ln_g.reshape(1, -1),
      ln_b.reshape(1, -1), jnp.asarray(_R_NP), jnp.asarray(_T_NP))


def _matmul_kernel(x_ref, w_ref, o_ref):
    o_ref[:, :] = jnp.dot(x_ref[:, :], w_ref[:, :],
                          preferred_element_type=jnp.float32,
                          precision=_HIGHEST)


def _matmul(x, w_t, block=512):
    n = x.shape[0]
    return pl.pallas_call(
        _matmul_kernel,
        grid=(n // block,),
        in_specs=[
            pl.BlockSpec((block, x.shape[1]), lambda i: (i, 0)),
            pl.BlockSpec(w_t.shape, lambda i: (0, 0)),
        ],
        out_specs=pl.BlockSpec((block, w_t.shape[1]), lambda i: (i, 0)),
        out_shape=jax.ShapeDtypeStruct((n, w_t.shape[1]), jnp.float32),
    )(x, w_t)


def kernel(x, Wq, c_keys, c_prime_keys, ln_g, ln_b, w_down, w_up, Wo):
    b, s, d = x.shape
    x2d = x.reshape(b * s, d)
    gidx, rw = _routing(x2d, Wq, c_keys, c_prime_keys, ln_g, ln_b)
    global_indices = gidx.reshape(b, s, N_HEAD, K)
    routing_weights = rw.reshape(b, s, N_HEAD, K)
    w_down_vals = jnp.take(w_down, global_indices, axis=0)
    w_up_vals = jnp.take(w_up, global_indices, axis=0)
    x_heads = x.reshape(b, s, N_HEAD, 1, HEAD_DIM)
    hidden = (x_heads * w_down_vals).sum(axis=-1)
    hidden = jax.nn.gelu(hidden, approximate=False)
    hidden = hidden * routing_weights
    out_heads = (hidden[..., None] * w_up_vals).sum(axis=3)
    out = _matmul(out_heads.reshape(b * s, d), Wo.T).reshape(b, s, d)
    return out


# speedup vs baseline: 12.8024x; 1.9528x over previous
import functools

import numpy as np
import jax
import jax.numpy as jnp
from jax import lax
from jax.experimental import pallas as pl
from jax.experimental.pallas import tpu as pltpu
from jax.experimental.pallas import tpu_sc as plsc

D_MODEL = 384
N_HEAD = 6
HEAD_DIM = 64
SUB = 32
NUM_EXPERTS = 65536
SQRT_N = 256
K = 16
B, S = 4, 2048

_HIGHEST = jax.lax.Precision.HIGHEST

_R_NP = np.zeros((16, 256), np.float32)
_T_NP = np.zeros((16, 256), np.float32)
for _a in range(16):
    for _b in range(16):
        _R_NP[_a, _a * 16 + _b] = 1.0
        _T_NP[_b, _a * 16 + _b] = 1.0

_NEG = np.float32(-3.0e38)
_BIG_I = np.int32(2 ** 30)


def _top16(s, payload=None):
    iota = jax.lax.broadcasted_iota(jnp.int32, s.shape, 1)
    work = s
    vals, idxs, pays = [], [], []
    for _ in range(16):
        m = jnp.max(work, axis=-1, keepdims=True)
        is_m = work == m
        pos = jnp.min(jnp.where(is_m, iota, _BIG_I), axis=-1, keepdims=True)
        sel = iota == pos
        vals.append(m)
        idxs.append(pos)
        if payload is not None:
            pays.append(jnp.sum(jnp.where(sel, payload, 0.0), axis=-1,
                                keepdims=True))
        work = jnp.where(sel, _NEG, work)
    vals = jnp.concatenate(vals, axis=1)
    idxs = jnp.concatenate(idxs, axis=1)
    pays = jnp.concatenate(pays, axis=1) if payload is not None else None
    return vals, idxs, pays


def _routing_kernel(x_ref, wq_ref, ck_ref, cpk_ref, g_ref, b_ref, r_ref,
                    t_ref, idx_ref, w_ref):
    xb = x_ref[:, :]
    q = jnp.dot(xb, wq_ref[:, :], preferred_element_type=jnp.float32)
    rm = r_ref[:, :]
    tm = t_ref[:, :]
    for h in range(N_HEAD):
        qh = q[:, h * HEAD_DIM:(h + 1) * HEAD_DIM]
        mu = jnp.mean(qh, axis=-1, keepdims=True)
        dq = qh - mu
        var = jnp.mean(dq * dq, axis=-1, keepdims=True)
        qn = dq / jnp.sqrt(var + 1e-5) * g_ref[0, :] + b_ref[0, :]
        s1 = jnp.dot(qn[:, :SUB], ck_ref[:, :],
                     preferred_element_type=jnp.float32)
        s2 = jnp.dot(qn[:, SUB:], cpk_ref[:, :],
                     preferred_element_type=jnp.float32)
        t1, i1, _ = _top16(s1)
        t2, i2, _ = _top16(s2)
        jv = (jnp.dot(t1, rm, preferred_element_type=jnp.float32,
                      precision=_HIGHEST)
              + jnp.dot(t2, tm, preferred_element_type=jnp.float32,
                        precision=_HIGHEST))
        jc = (jnp.dot(i1.astype(jnp.float32) * 256.0, rm,
                      preferred_element_type=jnp.float32, precision=_HIGHEST)
              + jnp.dot(i2.astype(jnp.float32), tm,
                        preferred_element_type=jnp.float32,
                        precision=_HIGHEST))
        fs, _, code = _top16(jv, payload=jc)
        m = jnp.max(fs, axis=-1, keepdims=True)
        e = jnp.exp(fs - m)
        w = e / jnp.sum(e, axis=-1, keepdims=True)
        idx_ref[:, h * K:(h + 1) * K] = code.astype(jnp.int32)
        w_ref[:, h * K:(h + 1) * K] = w


def _routing(x2d, Wq, c_keys, c_prime_keys, ln_g, ln_b, block=512):
    n = x2d.shape[0]
    grid = (n // block,)
    return pl.pallas_call(
        _routing_kernel,
        grid=grid,
        in_specs=[
            pl.BlockSpec((block, D_MODEL), lambda i: (i, 0)),
            pl.BlockSpec((D_MODEL, D_MODEL), lambda i: (0, 0)),
            pl.BlockSpec((SUB, SQRT_N), lambda i: (0, 0)),
            pl.BlockSpec((SUB, SQRT_N), lambda i: (0, 0)),
            pl.BlockSpec((1, HEAD_DIM), lambda i: (0, 0)),
            pl.BlockSpec((1, HEAD_DIM), lambda i: (0, 0)),
            pl.BlockSpec((16, 256), lambda i: (0, 0)),
            pl.BlockSpec((16, 256), lambda i: (0, 0)),
        ],
        out_specs=[
            pl.BlockSpec((block, N_HEAD * K), lambda i: (i, 0)),
            pl.BlockSpec((block, N_HEAD * K), lambda i: (i, 0)),
        ],
        out_shape=[
            jax.ShapeDtypeStruct((n, N_HEAD * K), jnp.int32),
            jax.ShapeDtypeStruct((n, N_HEAD * K), jnp.float32),
        ],
    )(x2d, Wq.T, c_keys.T, c_prime_keys.T, ln_g.reshape(1, -1),
      ln_b.reshape(1, -1), jnp.asarray(_R_NP), jnp.asarray(_T_NP))


_NT = 32
_TOK_PER_TILE = (B * S) // _NT
_C = 4
_NCHUNK = _TOK_PER_TILE // _C
_PAIRS = _C * N_HEAD
_ROWS = _PAIRS * K
_KW = N_HEAD * K


def _gather_body(idxf, wcat, cat_out, i0_v, i1_v, i2_v, cat_v, sem):
    wid = lax.axis_index("s") * 2 + lax.axis_index("c")
    t0_tile = wid * _TOK_PER_TILE
    idx_refs = (i0_v, i1_v, i2_v)

    def chunk_body(ci, carry):
        r0 = (t0_tile + ci * _C) * _KW
        for j in range(3):
            pltpu.sync_copy(idxf.at[pl.ds(r0 + j * 128, 128)], idx_refs[j])
        cps = []
        for j in range(3):
            cps.append(pltpu.async_copy(
                wcat.at[idx_refs[j]], cat_v.at[pl.ds(j * 128, 128)], sem))
        for cp in cps:
            cp.wait()
        pltpu.sync_copy(cat_v, cat_out.at[pl.ds(r0, _ROWS)])
        return carry

    lax.fori_loop(0, _NCHUNK, chunk_body, 0)


def _gather_rows(gidx, wcat):
    mesh = plsc.VectorSubcoreMesh(core_axis_name="c", subcore_axis_name="s")
    n = B * S * _KW
    fn = functools.partial(
        pl.kernel,
        mesh=mesh,
        out_type=jax.ShapeDtypeStruct((n, 2 * HEAD_DIM), jnp.float32),
        scratch_types=[
            pltpu.VMEM((128,), jnp.int32),
            pltpu.VMEM((128,), jnp.int32),
            pltpu.VMEM((128,), jnp.int32),
            pltpu.VMEM((_ROWS, 2 * HEAD_DIM), jnp.float32),
            pltpu.SemaphoreType.DMA,
        ],
    )(_gather_body)
    return fn(gidx.reshape(-1), wcat)


def _erf(u):
    ax = jnp.abs(u)
    t = 1.0 / (1.0 + 0.3275911 * ax)
    poly = t * (0.254829592 + t * (-0.284496736 + t * (
        1.421413741 + t * (-1.453152027 + t * 1.061405429))))
    e = 1.0 - poly * jnp.exp(-ax * ax)
    return jnp.where(u >= 0.0, e, -e)


def _expert_kernel(cat_ref, x_ref, w_ref, wo_ref, o_ref):
    xb = x_ref[:, :]
    wb = w_ref[:, :]
    accs = []
    for h in range(N_HEAD):
        xh = xb[:, h * HEAD_DIM:(h + 1) * HEAD_DIM]
        acc = jnp.zeros(xh.shape, jnp.float32)
        for kk in range(K):
            j = h * K + kk
            dnr = cat_ref[:, j, :HEAD_DIM]
            upr = cat_ref[:, j, HEAD_DIM:]
            hid = jnp.sum(xh * dnr, axis=-1, keepdims=True)
            u = hid * 0.7071067811865476
            geval = 0.5 * hid * (1.0 + _erf(u))
            gj = geval * wb[:, j:j + 1]
            acc = acc + gj * upr
        accs.append(acc)
    oh = jnp.concatenate(accs, axis=1)
    o_ref[:, :] = jnp.dot(oh, wo_ref[:, :],
                          preferred_element_type=jnp.float32)


def _expert_compute(cat3, x2d, rw, wo_t, block=128):
    n = x2d.shape[0]
    return pl.pallas_call(
        _expert_kernel,
        grid=(n // block,),
        in_specs=[
            pl.BlockSpec((block, _KW, 2 * HEAD_DIM), lambda i: (i, 0, 0)),
            pl.BlockSpec((block, D_MODEL), lambda i: (i, 0)),
            pl.BlockSpec((block, _KW), lambda i: (i, 0)),
            pl.BlockSpec((D_MODEL, D_MODEL), lambda i: (0, 0)),
        ],
        out_specs=pl.BlockSpec((block, D_MODEL), lambda i: (i, 0)),
        out_shape=jax.ShapeDtypeStruct((n, D_MODEL), jnp.float32),
    )(cat3, x2d, rw, wo_t)


def _matmul_kernel(x_ref, w_ref, o_ref):
    o_ref[:, :] = jnp.dot(x_ref[:, :], w_ref[:, :],
                          preferred_element_type=jnp.float32,
                          precision=_HIGHEST)


def _matmul(x, w_t, block=512):
    n = x.shape[0]
    return pl.pallas_call(
        _matmul_kernel,
        grid=(n // block,),
        in_specs=[
            pl.BlockSpec((block, x.shape[1]), lambda i: (i, 0)),
            pl.BlockSpec(w_t.shape, lambda i: (0, 0)),
        ],
        out_specs=pl.BlockSpec((block, w_t.shape[1]), lambda i: (i, 0)),
        out_shape=jax.ShapeDtypeStruct((n, w_t.shape[1]), jnp.float32),
    )(x, w_t)


def kernel(x, Wq, c_keys, c_prime_keys, ln_g, ln_b, w_down, w_up, Wo):
    b, s, d = x.shape
    x2d = x.reshape(b * s, d)
    gidx, rw = _routing(x2d, Wq, c_keys, c_prime_keys, ln_g, ln_b)
    wcat = jnp.concatenate([w_down, w_up], axis=1)
    cat_rows = _gather_rows(gidx, wcat)
    cat3 = cat_rows.reshape(b * s, _KW, 2 * HEAD_DIM)
    out = _expert_compute(cat3, x2d, rw, Wo.T).reshape(b, s, d)
    return out


# speedup vs baseline: 17.8799x; 1.3966x over previous
import functools

import numpy as np
import jax
import jax.numpy as jnp
from jax import lax
from jax.experimental import pallas as pl
from jax.experimental.pallas import tpu as pltpu
from jax.experimental.pallas import tpu_sc as plsc

D_MODEL = 384
N_HEAD = 6
HEAD_DIM = 64
SUB = 32
NUM_EXPERTS = 65536
SQRT_N = 256
K = 16
B, S = 4, 2048

_HIGHEST = jax.lax.Precision.HIGHEST

_R_NP = np.zeros((16, 256), np.float32)
_T_NP = np.zeros((16, 256), np.float32)
for _a in range(16):
    for _b in range(16):
        _R_NP[_a, _a * 16 + _b] = 1.0
        _T_NP[_b, _a * 16 + _b] = 1.0

_NEG = np.float32(-3.0e38)
_BIG_I = np.int32(2 ** 30)


def _top16(s, payload=None):
    iota = jax.lax.broadcasted_iota(jnp.int32, s.shape, 1)
    work = s
    vals, idxs, pays = [], [], []
    for _ in range(16):
        m = jnp.max(work, axis=-1, keepdims=True)
        is_m = work == m
        pos = jnp.min(jnp.where(is_m, iota, _BIG_I), axis=-1, keepdims=True)
        sel = iota == pos
        vals.append(m)
        idxs.append(pos)
        if payload is not None:
            pays.append(jnp.sum(jnp.where(sel, payload, 0.0), axis=-1,
                                keepdims=True))
        work = jnp.where(sel, _NEG, work)
    vals = jnp.concatenate(vals, axis=1)
    idxs = jnp.concatenate(idxs, axis=1)
    pays = jnp.concatenate(pays, axis=1) if payload is not None else None
    return vals, idxs, pays


def _routing_kernel(x_ref, wq_ref, ck_ref, cpk_ref, g_ref, b_ref, r_ref,
                    t_ref, idx_ref, w_ref):
    xb = x_ref[:, :]
    q = jnp.dot(xb, wq_ref[:, :], preferred_element_type=jnp.float32)
    rm = r_ref[:, :]
    tm = t_ref[:, :]
    for h in range(N_HEAD):
        qh = q[:, h * HEAD_DIM:(h + 1) * HEAD_DIM]
        mu = jnp.mean(qh, axis=-1, keepdims=True)
        dq = qh - mu
        var = jnp.mean(dq * dq, axis=-1, keepdims=True)
        qn = dq / jnp.sqrt(var + 1e-5) * g_ref[0, :] + b_ref[0, :]
        s1 = jnp.dot(qn[:, :SUB], ck_ref[:, :],
                     preferred_element_type=jnp.float32)
        s2 = jnp.dot(qn[:, SUB:], cpk_ref[:, :],
                     preferred_element_type=jnp.float32)
        t1, i1, _ = _top16(s1)
        t2, i2, _ = _top16(s2)
        jv = (jnp.dot(t1, rm, preferred_element_type=jnp.float32,
                      precision=_HIGHEST)
              + jnp.dot(t2, tm, preferred_element_type=jnp.float32,
                        precision=_HIGHEST))
        jc = (jnp.dot(i1.astype(jnp.float32) * 256.0, rm,
                      preferred_element_type=jnp.float32, precision=_HIGHEST)
              + jnp.dot(i2.astype(jnp.float32), tm,
                        preferred_element_type=jnp.float32,
                        precision=_HIGHEST))
        fs, _, code = _top16(jv, payload=jc)
        m = jnp.max(fs, axis=-1, keepdims=True)
        e = jnp.exp(fs - m)
        w = e / jnp.sum(e, axis=-1, keepdims=True)
        idx_ref[:, h * K:(h + 1) * K] = code.astype(jnp.int32)
        w_ref[:, h * K:(h + 1) * K] = w


def _routing(x2d, Wq, c_keys, c_prime_keys, ln_g, ln_b, block=512):
    n = x2d.shape[0]
    grid = (n // block,)
    return pl.pallas_call(
        _routing_kernel,
        grid=grid,
        in_specs=[
            pl.BlockSpec((block, D_MODEL), lambda i: (i, 0)),
            pl.BlockSpec((D_MODEL, D_MODEL), lambda i: (0, 0)),
            pl.BlockSpec((SUB, SQRT_N), lambda i: (0, 0)),
            pl.BlockSpec((SUB, SQRT_N), lambda i: (0, 0)),
            pl.BlockSpec((1, HEAD_DIM), lambda i: (0, 0)),
            pl.BlockSpec((1, HEAD_DIM), lambda i: (0, 0)),
            pl.BlockSpec((16, 256), lambda i: (0, 0)),
            pl.BlockSpec((16, 256), lambda i: (0, 0)),
        ],
        out_specs=[
            pl.BlockSpec((block, N_HEAD * K), lambda i: (i, 0)),
            pl.BlockSpec((block, N_HEAD * K), lambda i: (i, 0)),
        ],
        out_shape=[
            jax.ShapeDtypeStruct((n, N_HEAD * K), jnp.int32),
            jax.ShapeDtypeStruct((n, N_HEAD * K), jnp.float32),
        ],
    )(x2d, Wq.T, c_keys.T, c_prime_keys.T, ln_g.reshape(1, -1),
      ln_b.reshape(1, -1), jnp.asarray(_R_NP), jnp.asarray(_T_NP))


_NT = 32
_TOK_PER_TILE = (B * S) // _NT
_C = 4
_NCHUNK = _TOK_PER_TILE // _C
_PAIRS = _C * N_HEAD
_ROWS = _PAIRS * K
_KW = N_HEAD * K


def _gather_body(idxf, wcat, cat_out, i0_v, i1_v, i2_v, cat_v, sem):
    wid = lax.axis_index("s") * 2 + lax.axis_index("c")
    t0_tile = wid * _TOK_PER_TILE
    idx_refs = (i0_v, i1_v, i2_v)

    def chunk_body(ci, carry):
        r0 = (t0_tile + ci * _C) * _KW
        for j in range(3):
            pltpu.sync_copy(idxf.at[pl.ds(r0 + j * 128, 128)], idx_refs[j])
        cps = []
        for j in range(3):
            cps.append(pltpu.async_copy(
                wcat.at[idx_refs[j]], cat_v.at[pl.ds(j * 128, 128)], sem))
        for cp in cps:
            cp.wait()
        pltpu.sync_copy(cat_v, cat_out.at[pl.ds(r0, _ROWS)])
        return carry

    lax.fori_loop(0, _NCHUNK, chunk_body, 0)


def _gather_rows(gidx, wcat):
    mesh = plsc.VectorSubcoreMesh(core_axis_name="c", subcore_axis_name="s")
    n = B * S * _KW
    fn = functools.partial(
        pl.kernel,
        mesh=mesh,
        out_type=jax.ShapeDtypeStruct((n, 2 * HEAD_DIM), jnp.float32),
        scratch_types=[
            pltpu.VMEM((128,), jnp.int32),
            pltpu.VMEM((128,), jnp.int32),
            pltpu.VMEM((128,), jnp.int32),
            pltpu.VMEM((_ROWS, 2 * HEAD_DIM), jnp.float32),
            pltpu.SemaphoreType.DMA,
        ],
    )(_gather_body)
    return fn(gidx.reshape(-1), wcat)


def _erf(u):
    ax = jnp.abs(u)
    t = 1.0 / (1.0 + 0.3275911 * ax)
    poly = t * (0.254829592 + t * (-0.284496736 + t * (
        1.421413741 + t * (-1.453152027 + t * 1.061405429))))
    e = 1.0 - poly * jnp.exp(-ax * ax)
    return jnp.where(u >= 0.0, e, -e)


def _expert_kernel(cat_ref, x3_ref, w_ref, wo_ref, o_ref):
    x3 = x3_ref[:, :, :]
    wb = w_ref[:, :]
    hids = []
    for h in range(N_HEAD):
        prod = (cat_ref[:, h * K:(h + 1) * K, :HEAD_DIM]
                * x3[:, h:h + 1, :])
        hids.append(jnp.sum(prod, axis=-1))
    hid = jnp.concatenate(hids, axis=1)
    u = hid * 0.7071067811865476
    g = 0.5 * hid * (1.0 + _erf(u)) * wb
    accs = []
    for h in range(N_HEAD):
        acc = jnp.zeros((x3.shape[0], HEAD_DIM), jnp.float32)
        for kk in range(K):
            j = h * K + kk
            acc = acc + g[:, j:j + 1] * cat_ref[:, j, HEAD_DIM:]
        accs.append(acc)
    oh = jnp.concatenate(accs, axis=1)
    o_ref[:, :] = jnp.dot(oh, wo_ref[:, :],
                          preferred_element_type=jnp.float32)


def _expert_compute(cat3, x2d, rw, wo_t, block=64):
    n = x2d.shape[0]
    return pl.pallas_call(
        _expert_kernel,
        grid=(n // block,),
        in_specs=[
            pl.BlockSpec((block, _KW, 2 * HEAD_DIM), lambda i: (i, 0, 0)),
            pl.BlockSpec((block, N_HEAD, HEAD_DIM), lambda i: (i, 0, 0)),
            pl.BlockSpec((block, _KW), lambda i: (i, 0)),
            pl.BlockSpec((D_MODEL, D_MODEL), lambda i: (0, 0)),
        ],
        out_specs=pl.BlockSpec((block, D_MODEL), lambda i: (i, 0)),
        out_shape=jax.ShapeDtypeStruct((n, D_MODEL), jnp.float32),
    )(cat3, x2d.reshape(-1, N_HEAD, HEAD_DIM), rw, wo_t)


def _matmul_kernel(x_ref, w_ref, o_ref):
    o_ref[:, :] = jnp.dot(x_ref[:, :], w_ref[:, :],
                          preferred_element_type=jnp.float32,
                          precision=_HIGHEST)


def _matmul(x, w_t, block=512):
    n = x.shape[0]
    return pl.pallas_call(
        _matmul_kernel,
        grid=(n // block,),
        in_specs=[
            pl.BlockSpec((block, x.shape[1]), lambda i: (i, 0)),
            pl.BlockSpec(w_t.shape, lambda i: (0, 0)),
        ],
        out_specs=pl.BlockSpec((block, w_t.shape[1]), lambda i: (i, 0)),
        out_shape=jax.ShapeDtypeStruct((n, w_t.shape[1]), jnp.float32),
    )(x, w_t)


def kernel(x, Wq, c_keys, c_prime_keys, ln_g, ln_b, w_down, w_up, Wo):
    b, s, d = x.shape
    x2d = x.reshape(b * s, d)
    gidx, rw = _routing(x2d, Wq, c_keys, c_prime_keys, ln_g, ln_b)
    wcat = jnp.concatenate([w_down, w_up], axis=1)
    cat_rows = _gather_rows(gidx, wcat)
    cat3 = cat_rows.reshape(b * s, _KW, 2 * HEAD_DIM)
    out = _expert_compute(cat3, x2d, rw, Wo.T).reshape(b, s, d)
    return out
